# Initial kernel scaffold; baseline (speedup 1.0000x reference)
import jax
import jax.numpy as jnp
from jax.experimental import pallas as pl


def kernel(x, Wg, W1, b1, W2, b2):
    raise NotImplementedError("write your pallas kernel here")


# speedup vs baseline: 1.5936x; 1.5936x over previous
import functools
import numpy as np
import jax
import jax.numpy as jnp
from jax.experimental import pallas as pl
from jax.experimental.pallas import tpu as pltpu


def _route_body(C, E, x_ref, wg_ref, slot_ref, w_ref):
    N = x_ref.shape[0]
    xf = x_ref[...]
    logits = jnp.dot(xf, wg_ref[...], preferred_element_type=jnp.float32)
    m = jnp.max(logits, axis=1, keepdims=True)
    gv = 1.0 / jnp.sum(jnp.exp(logits - m), axis=1, keepdims=True)
    e_iota = jax.lax.broadcasted_iota(jnp.int32, logits.shape, 1)
    idx = jnp.min(jnp.where(logits == m, e_iota, E), axis=1, keepdims=True)
    oh = (e_iota == idx).astype(jnp.float32)
    r = jax.lax.broadcasted_iota(jnp.int32, (N, N), 0)
    c = jax.lax.broadcasted_iota(jnp.int32, (N, N), 1)
    tri = (c < r).astype(jnp.float32)
    cum = jnp.dot(tri, oh, preferred_element_type=jnp.float32)
    pos = jnp.sum(cum * oh, axis=1, keepdims=True).astype(jnp.int32)
    keepm = pos < C
    slot_ref[...] = jnp.where(keepm, idx * C + pos, E * C)
    w_ref[...] = jnp.where(keepm, gv, 0.0)


def _expert_body(C, E, slot_ref, w_ref, x_ref, w1_ref, b1_ref, w2_ref,
                 b2_ref, out_ref, s_acc):
    e = pl.program_id(0)
    N = slot_ref.shape[0]
    slot_col = slot_ref[...]
    c_iota = jax.lax.broadcasted_iota(jnp.int32, (N, C), 1)
    P = (slot_col == e * C + c_iota).astype(jnp.float32)
    dispx = jax.lax.dot_general(
        P, x_ref[...], (((0,), (0,)), ((), ())),
        preferred_element_type=jnp.float32)
    h = jnp.maximum(
        jnp.dot(dispx, w1_ref[0], preferred_element_type=jnp.float32)
        + b1_ref[0], 0.0)
    w2s = jnp.sum(w2_ref[0], axis=1, keepdims=True)
    val = jnp.dot(h, w2s, preferred_element_type=jnp.float32) \
        + jnp.sum(b2_ref[0])
    contrib = jnp.dot(P, val, preferred_element_type=jnp.float32) \
        * w_ref[...]

    @pl.when(e == 0)
    def _():
        s_acc[...] = contrib

    @pl.when(e > 0)
    def _():
        s_acc[...] = s_acc[...] + contrib

    @pl.when(e == E - 1)
    def _():
        s = s_acc[...]
        mx = jnp.max(s, axis=0, keepdims=True)
        lse = jnp.log(jnp.sum(jnp.exp(s - mx), axis=0, keepdims=True)) + mx
        out_ref[...] = s - lse


def kernel(x, Wg, W1, b1, W2, b2):
    B_, T_, D_ = x.shape
    N = B_ * T_
    E_ = Wg.shape[1]
    H_ = W1.shape[2]
    C = int(np.ceil(N * 1.25 / E_))
    xf = x.reshape(N, D_)

    slot, w = pl.pallas_call(
        functools.partial(_route_body, C, E_),
        out_shape=[jax.ShapeDtypeStruct((N, 1), jnp.int32),
                   jax.ShapeDtypeStruct((N, 1), jnp.float32)],
    )(xf, Wg)

    out = pl.pallas_call(
        functools.partial(_expert_body, C, E_),
        grid=(E_,),
        in_specs=[
            pl.BlockSpec((N, 1), lambda e: (0, 0)),
            pl.BlockSpec((N, 1), lambda e: (0, 0)),
            pl.BlockSpec((N, D_), lambda e: (0, 0)),
            pl.BlockSpec((1, D_, H_), lambda e: (e, 0, 0)),
            pl.BlockSpec((1, 1, H_), lambda e: (e, 0, 0)),
            pl.BlockSpec((1, H_, D_), lambda e: (e, 0, 0)),
            pl.BlockSpec((1, 1, D_), lambda e: (e, 0, 0)),
        ],
        out_specs=pl.BlockSpec((N, 1), lambda e: (0, 0)),
        out_shape=jax.ShapeDtypeStruct((N, 1), jnp.float32),
        scratch_shapes=[pltpu.VMEM((N, 1), jnp.float32)],
    )(slot, w, xf, W1, b1.reshape(E_, 1, H_), W2, b2.reshape(E_, 1, D_))
    return out.reshape(B_, T_)
